# Initial kernel scaffold; baseline (speedup 1.0000x reference)
import jax
import jax.numpy as jnp
from jax.experimental import pallas as pl


def kernel(x, W_enc, b_enc, W_dec, b_dec):
    raise NotImplementedError("write your pallas kernel here")


# speedup vs baseline: 1.6672x; 1.6672x over previous
import jax
import jax.numpy as jnp
from jax.experimental import pallas as pl
from jax.experimental.pallas import tpu as pltpu

B = 32
D_IN = 4096
D_HID = 32768
K = 64
BLK_E = 2048
BLK_D = 2048


def _encode_body(x_ref, w_ref, b_ref, r_ref):
    acc = jax.lax.dot_general(
        x_ref[...], w_ref[...], (((1,), (1,)), ((), ())),
        preferred_element_type=jnp.float32)
    pre = acc + b_ref[...].astype(jnp.float32)
    r_ref[...] = jnp.maximum(pre, 0.0)


def _key(r, col0, blk):
    bits = jax.lax.bitcast_convert_type(r, jnp.int32)
    col = col0 + jax.lax.broadcasted_iota(jnp.int32, (B, blk), 1)
    return (bits & jnp.int32(-65536)) | (D_HID - 1 - col)


def _thresh_body(r_ref, tv_ref):
    keys = _key(r_ref[...], 0, D_HID)

    def vstep(_, lohi):
        lo, hi = lohi
        mid = lo + ((hi - lo) >> 1)
        cnt = jnp.sum((keys >= mid).astype(jnp.int32), axis=1, keepdims=True)
        ge = cnt >= K
        return (jnp.where(ge, mid, lo), jnp.where(ge, hi, mid))

    lo0 = jnp.zeros((B, 1), jnp.int32)
    hi0 = jnp.full((B, 1), 0x7FFFFFFF, jnp.int32)
    vb, _ = jax.lax.fori_loop(0, 31, vstep, (lo0, hi0))
    tv_ref[...] = jnp.broadcast_to(vb, (B, 128))


def _decode_body(r_ref, tv_ref, w_ref, b_ref, f_ref, o_ref, acc_ref):
    k = pl.program_id(0)
    r = r_ref[...]
    mask = _key(r, k * BLK_D, BLK_D) >= tv_ref[:, 0:1]
    feats = jnp.where(mask, r.astype(jnp.bfloat16), jnp.bfloat16(0))
    f_ref[...] = feats
    part = jax.lax.dot_general(
        feats, w_ref[...], (((1,), (1,)), ((), ())),
        preferred_element_type=jnp.float32)

    @pl.when(k == 0)
    def _():
        acc_ref[...] = part

    @pl.when(k > 0)
    def _():
        acc_ref[...] += part

    @pl.when(k == pl.num_programs(0) - 1)
    def _():
        o_ref[...] = (acc_ref[...] + b_ref[...].astype(jnp.float32)
                      ).astype(jnp.bfloat16)


def kernel(x, W_enc, b_enc, W_dec, b_dec):
    r = pl.pallas_call(
        _encode_body,
        grid=(D_HID // BLK_E,),
        in_specs=[
            pl.BlockSpec((B, D_IN), lambda k: (0, 0)),
            pl.BlockSpec((BLK_E, D_IN), lambda k: (k, 0)),
            pl.BlockSpec((1, BLK_E), lambda k: (0, k)),
        ],
        out_specs=pl.BlockSpec((B, BLK_E), lambda k: (0, k)),
        out_shape=jax.ShapeDtypeStruct((B, D_HID), jnp.float32),
    )(x, W_enc, b_enc.reshape(1, D_HID))

    tv = pl.pallas_call(
        _thresh_body,
        out_shape=jax.ShapeDtypeStruct((B, 128), jnp.int32),
    )(r)

    f, o = pl.pallas_call(
        _decode_body,
        grid=(D_HID // BLK_D,),
        in_specs=[
            pl.BlockSpec((B, BLK_D), lambda k: (0, k)),
            pl.BlockSpec((B, 128), lambda k: (0, 0)),
            pl.BlockSpec((D_IN, BLK_D), lambda k: (0, k)),
            pl.BlockSpec((1, D_IN), lambda k: (0, 0)),
        ],
        out_specs=[
            pl.BlockSpec((B, BLK_D), lambda k: (0, k)),
            pl.BlockSpec((B, D_IN), lambda k: (0, 0)),
        ],
        out_shape=[
            jax.ShapeDtypeStruct((B, D_HID), jnp.bfloat16),
            jax.ShapeDtypeStruct((B, D_IN), jnp.bfloat16),
        ],
        scratch_shapes=[pltpu.VMEM((B, D_IN), jnp.float32)],
    )(r, tv, W_dec, b_dec.reshape(1, D_IN))

    return (o, f)
